# Initial kernel scaffold; baseline (speedup 1.0000x reference)
import jax
import jax.numpy as jnp
from jax.experimental import pallas as pl


def kernel(x, edge_index, edge_attr, params):
    raise NotImplementedError("write your pallas kernel here")


# speedup vs baseline: 13.5363x; 13.5363x over previous
import functools
from functools import partial

import jax
import jax.numpy as jnp
import numpy as np
from jax.experimental import pallas as pl
from jax.experimental.pallas import tpu as pltpu

N_NODES = 10000
N_EDGES = 320000
D_EDGE = 16
NUM_TOWERS_HID = 4
AVG_DEG_LOG = float(np.log(33.0))
BN_EPS = 1e-5

NODE_BLK = 1000


def _prep_conv(p, towers, f_in, f_out):
    W = towers * f_in
    f_out_t = f_out // towers
    W_pre = p['W_pre']
    w1 = jnp.transpose(W_pre[:, :f_in, :], (1, 0, 2)).reshape(f_in, W)
    w2 = jnp.transpose(W_pre[:, f_in:2 * f_in, :], (1, 0, 2)).reshape(f_in, W)
    w3 = jnp.transpose(W_pre[:, 2 * f_in:, :], (1, 0, 2)).reshape(f_in, W)
    wc = p['W_edge'] @ w3
    c0 = p['b_edge'] @ w3
    b_pre = p['b_pre'].reshape(W)

    W_post = p['W_post']
    px = jnp.transpose(W_post[:, :f_in, :], (1, 0, 2)).reshape(f_in, f_out)

    def blockdiag(k):
        blk = W_post[:, (1 + k) * f_in:(2 + k) * f_in, :]
        eye = jnp.eye(towers, dtype=blk.dtype)
        bd = jnp.einsum('tio,ts->tiso', blk, eye).reshape(W, f_out)
        return bd

    p_id = jnp.concatenate([blockdiag(k) for k in range(4)], axis=0)
    p_amp = jnp.concatenate([blockdiag(4 + k) for k in range(4)], axis=0)
    p_att = jnp.concatenate([blockdiag(8 + k) for k in range(4)], axis=0)
    b_post = p['b_post'].reshape(f_out)
    return dict(w1=w1, w2=w2, wc=wc, c0=c0, b_pre=b_pre,
                px=px, p_id=p_id, p_amp=p_amp, p_att=p_att, b_post=b_post,
                W_lin=p['W_lin'], b_lin=p['b_lin'], W=W, f_in=f_in, f_out=f_out)


def _post_kernel(sum_ref, sq_ref, mx_ref, mn_ref, deg_ref, base_ref, x_ref,
                 px_ref, pid_ref, pamp_ref, patt_ref, wlin_ref,
                 bpost_ref, blin_ref, gamma_ref, beta_ref,
                 out_ref, *, do_bn_relu, do_logsoftmax):
    deg = deg_ref[...]
    base = base_ref[...]
    deg_safe = jnp.maximum(deg, 1.0)
    inv = 1.0 / deg_safe
    has = deg > 0.0
    meanD = sum_ref[...] * inv
    mean = jnp.where(has, base + meanD, 0.0)
    var = sq_ref[...] * inv - meanD * meanD
    std = jnp.sqrt(jnp.maximum(var, 0.0) + 1e-5)
    mx = jnp.where(has, base + mx_ref[...], 0.0)
    mn = jnp.where(has, base + mn_ref[...], 0.0)
    g = jnp.concatenate([mean, mn, mx, std], axis=1)
    logd = jnp.log(deg + 1.0)
    amp = logd * (1.0 / AVG_DEG_LOG)
    att = jnp.where(has, AVG_DEG_LOG / jnp.maximum(logd, 1e-12), 1.0)
    y = (x_ref[...] @ px_ref[...]
         + g @ pid_ref[...]
         + amp * (g @ pamp_ref[...])
         + att * (g @ patt_ref[...])
         + bpost_ref[...])
    y = y @ wlin_ref[...] + blin_ref[...]
    if do_bn_relu:
        scale = gamma_ref[...] * (1.0 / np.sqrt(1.0 + BN_EPS))
        y = jnp.maximum(y * scale + beta_ref[...], 0.0)
    if do_logsoftmax:
        y = y - jnp.max(y, axis=1, keepdims=True)
        y = y - jnp.log(jnp.sum(jnp.exp(y), axis=1, keepdims=True))
    out_ref[...] = y


def _post_stage(sumD, sqD, mxD, mnD, deg, base, x, w, gamma, beta,
                do_bn_relu, do_logsoftmax):
    N = x.shape[0]
    W = w['W']
    f_in = w['f_in']
    f_out = w['f_out']
    grid = N // NODE_BLK
    nb = lambda width: pl.BlockSpec((NODE_BLK, width), lambda i: (i, 0))
    full = lambda a, b: pl.BlockSpec((a, b), lambda i: (0, 0))
    gamma2 = gamma.reshape(1, f_out)
    beta2 = beta.reshape(1, f_out)
    out = pl.pallas_call(
        partial(_post_kernel, do_bn_relu=do_bn_relu,
                do_logsoftmax=do_logsoftmax),
        grid=(grid,),
        in_specs=[nb(W), nb(W), nb(W), nb(W), nb(1), nb(W), nb(f_in),
                  full(f_in, f_out), full(4 * W, f_out), full(4 * W, f_out),
                  full(4 * W, f_out), full(f_out, f_out),
                  full(1, f_out), full(1, f_out), full(1, f_out),
                  full(1, f_out)],
        out_specs=nb(f_out),
        out_shape=jax.ShapeDtypeStruct((N, f_out), jnp.float32),
    )(sumD, sqD, mxD, mnD, deg.reshape(N, 1), base, x,
      w['px'], w['p_id'], w['p_amp'], w['p_att'], w['W_lin'],
      w['b_post'].reshape(1, f_out), w['b_lin'].reshape(1, f_out),
      gamma2, beta2)
    return out


def _conv_layer(x, src, dst, edge_attr, deg, w, gamma, beta,
                do_bn_relu, do_logsoftmax):
    N = x.shape[0]
    base = x @ w['w1'] + w['b_pre']
    B = x @ w['w2']
    C = edge_attr @ w['wc'] + w['c0']
    D = B[src] + C
    sumD = jax.ops.segment_sum(D, dst, num_segments=N)
    sqD = jax.ops.segment_sum(D * D, dst, num_segments=N)
    mxD = jax.ops.segment_max(D, dst, num_segments=N)
    mnD = jax.ops.segment_min(D, dst, num_segments=N)
    has = (deg > 0)[:, None]
    mxD = jnp.where(has, mxD, 0.0)
    mnD = jnp.where(has, mnD, 0.0)
    return _post_stage(sumD, sqD, mxD, mnD, deg, base, x, w,
                       gamma, beta, do_bn_relu, do_logsoftmax)


def kernel(x, edge_index, edge_attr, params):
    src = edge_index[0]
    dst = edge_index[1]
    N = x.shape[0]
    f0 = x.shape[1]
    c0p = params['conv0']
    c1p = params['conv1']
    c2p = params['conv_out']
    f_hid = c1p['W_lin'].shape[0]
    n_cls = c2p['W_lin'].shape[0]
    w0 = _prep_conv(c0p, NUM_TOWERS_HID, f0, f_hid)
    w1 = _prep_conv(c1p, NUM_TOWERS_HID, f_hid, f_hid)
    w2 = _prep_conv(c2p, 1, f_hid, n_cls)

    ones = jnp.ones((src.shape[0],), dtype=x.dtype)
    deg = jax.ops.segment_sum(ones, dst, num_segments=N)

    h = _conv_layer(x, src, dst, edge_attr, deg, w0,
                    params['bn0']['gamma'], params['bn0']['beta'],
                    True, False)
    h = _conv_layer(h, src, dst, edge_attr, deg, w1,
                    params['bn1']['gamma'], params['bn1']['beta'],
                    True, False)
    zeros = jnp.zeros((n_cls,), dtype=x.dtype)
    h = _conv_layer(h, src, dst, edge_attr, deg, w2,
                    zeros + 1.0, zeros, False, True)
    return h


# speedup vs baseline: 37.7387x; 2.7880x over previous
import functools
from functools import partial

import jax
import jax.numpy as jnp
import numpy as np
from jax import lax
from jax.experimental import pallas as pl
from jax.experimental.pallas import tpu as pltpu
from jax.experimental.pallas import tpu_sc as plsc

N_NODES = 10000
D_EDGE = 16
NUM_TOWERS_HID = 4
AVG_DEG_LOG = float(np.log(33.0))
BN_EPS = 1e-5

N_PAD = 10752
NODES_PER_RANGE = 112
RANGES_PER_TILE = 3
N_TILES = 32
OFF_LEN = N_PAD + 16
EDGE_BLK = 2048
K_EDGE = 128
NODE_BLK = 512
NEG_INIT = float('-inf')


def _prep_conv(p, towers, f_in, f_out):
    W = towers * f_in
    W_pre = p['W_pre']
    w1 = jnp.transpose(W_pre[:, :f_in, :], (1, 0, 2)).reshape(f_in, W)
    w2 = jnp.transpose(W_pre[:, f_in:2 * f_in, :], (1, 0, 2)).reshape(f_in, W)
    w3 = jnp.transpose(W_pre[:, 2 * f_in:, :], (1, 0, 2)).reshape(f_in, W)
    wc = p['W_edge'] @ w3
    c0 = p['b_edge'] @ w3
    b_pre = p['b_pre'].reshape(W)

    W_post = p['W_post']
    px = jnp.transpose(W_post[:, :f_in, :], (1, 0, 2)).reshape(f_in, f_out)

    def blockdiag(k):
        blk = W_post[:, (1 + k) * f_in:(2 + k) * f_in, :]
        eye = jnp.eye(towers, dtype=blk.dtype)
        return jnp.einsum('tio,ts->tiso', blk, eye).reshape(W, f_out)

    p_id = jnp.concatenate([blockdiag(k) for k in range(4)], axis=0)
    p_amp = jnp.concatenate([blockdiag(4 + k) for k in range(4)], axis=0)
    p_att = jnp.concatenate([blockdiag(8 + k) for k in range(4)], axis=0)
    W_sc = max(W, 128)
    if W_sc != W:
        padw = lambda a: jnp.concatenate(
            [a, jnp.zeros((a.shape[0], W_sc - W), a.dtype)], axis=1)
        w2, wc = padw(w2), padw(wc)
        c0 = jnp.concatenate([c0, jnp.zeros((W_sc - W,), c0.dtype)])
    return dict(w1=w1, w2=w2, wc=wc, c0=c0, b_pre=b_pre,
                px=px, p_id=p_id, p_amp=p_amp, p_att=p_att,
                b_post=p['b_post'].reshape(f_out),
                W_lin=p['W_lin'], b_lin=p['b_lin'],
                W=W, W_sc=W_sc, f_in=f_in, f_out=f_out)


def _node_mm_kernel(x_ref, w1_ref, w2_ref, bpre_ref, base_ref, b_ref):
    x = x_ref[...]
    base_ref[...] = jnp.dot(x, w1_ref[...],
                            preferred_element_type=jnp.float32) + bpre_ref[...]
    b_ref[...] = jnp.dot(x, w2_ref[...], preferred_element_type=jnp.float32)


def _node_mm(x, w):
    n = x.shape[0]
    f_in, W, W_sc = w['f_in'], w['W'], w['W_sc']
    nb = lambda width: pl.BlockSpec((NODE_BLK, width), lambda i: (i, 0))
    full = lambda a, b: pl.BlockSpec((a, b), lambda i: (0, 0))
    return pl.pallas_call(
        _node_mm_kernel,
        grid=(n // NODE_BLK,),
        in_specs=[nb(f_in), full(f_in, W), full(f_in, W_sc), full(1, W)],
        out_specs=[nb(W), nb(W_sc)],
        out_shape=[jax.ShapeDtypeStruct((n, W), jnp.float32),
                   jax.ShapeDtypeStruct((n, W_sc), jnp.float32)],
    )(x, w['w1'], w['w2'], w['b_pre'].reshape(1, W))


def _edge_mm_kernel(ea_ref, wc_ref, c0_ref, c_ref):
    c_ref[...] = jnp.dot(ea_ref[...], wc_ref[...],
                         preferred_element_type=jnp.float32) + c0_ref[...]


def _edge_mm(ea_pad, w):
    e_pad = ea_pad.shape[0]
    W = w['W_sc']
    return pl.pallas_call(
        _edge_mm_kernel,
        grid=(e_pad // EDGE_BLK,),
        in_specs=[pl.BlockSpec((EDGE_BLK, D_EDGE), lambda i: (i, 0)),
                  pl.BlockSpec((D_EDGE, W), lambda i: (0, 0)),
                  pl.BlockSpec((1, W), lambda i: (0, 0))],
        out_specs=pl.BlockSpec((EDGE_BLK, W), lambda i: (i, 0)),
        out_shape=jax.ShapeDtypeStruct((e_pad, W), jnp.float32),
    )(ea_pad, w['wc'], w['c0'].reshape(1, W))


def _sc_segment_stats(B, C, src_s, perm_s, dst_s, off, W):
    WC = min(W, 128)
    n_cc = W // WC
    n_g = WC // 16
    NT = NODES_PER_RANGE + 1
    full_w = (n_cc == 1)

    mesh = plsc.VectorSubcoreMesh(core_axis_name="c", subcore_axis_name="s")

    @functools.partial(
        pl.kernel, mesh=mesh,
        out_type=[jax.ShapeDtypeStruct((N_PAD, W), jnp.float32)] * 4,
        scratch_types=[
            pltpu.VMEM((NODES_PER_RANGE + 16,), jnp.int32),
            pltpu.VMEM((2, K_EDGE), jnp.int32),
            pltpu.VMEM((2, K_EDGE), jnp.int32),
            pltpu.VMEM((2, K_EDGE), jnp.int32),
            pltpu.SMEM((2, K_EDGE), jnp.int32),
            pltpu.VMEM((2, K_EDGE, WC), jnp.float32),
            pltpu.VMEM((2, K_EDGE, WC), jnp.float32),
            pltpu.VMEM((NT, WC), jnp.float32),
            pltpu.VMEM((NT, WC), jnp.float32),
            pltpu.VMEM((NT, WC), jnp.float32),
            pltpu.VMEM((NT, WC), jnp.float32),
            pltpu.SemaphoreType.DMA,
            pltpu.SemaphoreType.DMA,
            pltpu.SemaphoreType.DMA,
            pltpu.SemaphoreType.DMA,
        ],
    )
    def seg_kernel(b_h, c_h, src_h, perm_h, dst_h, off_h,
                   o_sum, o_sq, o_mx, o_mn,
                   offv, idxv, permv, dstv, rowsv, bbuf, cbuf,
                   asum, asq, amx, amn, sem_b0, sem_b1, sem_c0, sem_c1):
        cid = lax.axis_index("c")
        sid = lax.axis_index("s")
        t = sid * 2 + cid
        sem_b = (sem_b0, sem_b1)
        sem_c = (sem_c0, sem_c1)
        lane = lax.iota(jnp.int32, 16)

        def do_range(r, _):
            rng = t + r * N_TILES
            n0 = pl.multiple_of(rng * NODES_PER_RANGE, 8)
            pltpu.sync_copy(off_h.at[pl.ds(n0, NODES_PER_RANGE + 16)], offv)
            e0 = offv[pl.ds(0, 16)][0]
            e1 = offv[pl.ds(NODES_PER_RANGE, 16)][0]
            e0a = lax.bitwise_and(e0, jnp.int32(-8))
            nch = lax.div(e1 - e0a + jnp.int32(K_EDGE - 1),
                          jnp.int32(K_EDGE))

            def col_pass(cc, _):
                col0 = pl.multiple_of(cc * WC, 128)
                csl = slice(None) if full_w else pl.ds(col0, WC)

                def load_chunk(i, b):
                    base = pl.multiple_of(e0a + i * K_EDGE, 8)
                    pltpu.sync_copy(src_h.at[pl.ds(base, K_EDGE)],
                                    idxv.at[b])
                    pltpu.sync_copy(perm_h.at[pl.ds(base, K_EDGE)],
                                    permv.at[b])
                    pltpu.sync_copy(dst_h.at[pl.ds(base, K_EDGE)],
                                    dstv.at[b])
                    pltpu.async_copy(b_h.at[idxv.at[b], csl], bbuf.at[b],
                                     sem_b[b])
                    pltpu.async_copy(c_h.at[permv.at[b], csl], cbuf.at[b],
                                     sem_c[b])
                    for j in range(K_EDGE // 16):
                        eidx = base + j * 16 + lane
                        dv = dstv[b, pl.ds(j * 16, 16)]
                        ok = (eidx >= e0) & (eidx < e1)
                        rows = jnp.where(ok, dv - n0, jnp.int32(NT - 1))
                        for k in range(16):
                            rowsv[b, j * 16 + k] = rows[k]

                def init_row(r, _):
                    for g in range(n_g):
                        sl = pl.ds(g * 16, 16)
                        asum[r, sl] = jnp.zeros((16,), jnp.float32)
                        asq[r, sl] = jnp.zeros((16,), jnp.float32)
                        amx[r, sl] = jnp.full((16,), NEG_INIT, jnp.float32)
                        amn[r, sl] = jnp.full((16,), -NEG_INIT, jnp.float32)
                    return 0
                lax.fori_loop(0, NT, init_row, 0)

                @pl.when(nch > 0)
                def _():
                    load_chunk(0, 0)

                def outer(i2, _):
                    for b in range(2):
                        i = i2 * 2 + b

                        @pl.when(i < nch)
                        def _():
                            @pl.when(i + 1 < nch)
                            def _():
                                load_chunk(i + 1, (b + 1) % 2)
                            pltpu.make_async_copy(
                                b_h.at[idxv.at[b], csl], bbuf.at[b],
                                sem_b[b]).wait()
                            pltpu.make_async_copy(
                                c_h.at[permv.at[b], csl], cbuf.at[b],
                                sem_c[b]).wait()

                            def acc_body(ke, _):
                                row = rowsv[b, ke]
                                for g in range(n_g):
                                    sl = pl.ds(g * 16, 16)
                                    d = bbuf[b, ke, sl] + cbuf[b, ke, sl]
                                    plsc.addupdate(asum.at[row, sl], d)
                                    plsc.addupdate(asq.at[row, sl], d * d)
                                    amx[row, sl] = jnp.maximum(
                                        amx[row, sl], d)
                                    amn[row, sl] = jnp.minimum(
                                        amn[row, sl], d)
                                return 0
                            lax.fori_loop(0, K_EDGE, acc_body, 0)
                    return 0
                lax.fori_loop(0, lax.div(nch + 1, jnp.int32(2)), outer, 0)

                nsl = pl.ds(n0, NODES_PER_RANGE)
                src_sl = pl.ds(0, NODES_PER_RANGE)
                pltpu.sync_copy(asum.at[src_sl, :], o_sum.at[nsl, csl])
                pltpu.sync_copy(asq.at[src_sl, :], o_sq.at[nsl, csl])
                pltpu.sync_copy(amx.at[src_sl, :], o_mx.at[nsl, csl])
                pltpu.sync_copy(amn.at[src_sl, :], o_mn.at[nsl, csl])
                return 0
            lax.fori_loop(0, n_cc, col_pass, 0)
            return 0
        lax.fori_loop(0, RANGES_PER_TILE, do_range, 0)

    return seg_kernel(B, C, src_s, perm_s, dst_s, off)


def _post_kernel(sum_ref, sq_ref, mx_ref, mn_ref, deg_ref, base_ref, x_ref,
                 px_ref, pid_ref, pamp_ref, patt_ref, wlin_ref,
                 bpost_ref, blin_ref, gamma_ref, beta_ref,
                 out_ref, *, do_bn_relu, do_logsoftmax):
    deg = deg_ref[...]
    base = base_ref[...]
    deg_safe = jnp.maximum(deg, 1.0)
    inv = 1.0 / deg_safe
    has = deg > 0.0
    meanD = sum_ref[...] * inv
    mean = jnp.where(has, base + meanD, 0.0)
    var = sq_ref[...] * inv - meanD * meanD
    std = jnp.sqrt(jnp.maximum(var, 0.0) + 1e-5)
    mx = jnp.where(has, base + mx_ref[...], 0.0)
    mn = jnp.where(has, base + mn_ref[...], 0.0)
    g = jnp.concatenate([mean, mn, mx, std], axis=1)
    logd = jnp.log(deg + 1.0)
    amp = logd * (1.0 / AVG_DEG_LOG)
    att = jnp.where(has, AVG_DEG_LOG / jnp.maximum(logd, 1e-12), 1.0)
    dot = lambda a, b: jnp.dot(a, b, preferred_element_type=jnp.float32)
    y = (dot(x_ref[...], px_ref[...])
         + dot(g, pid_ref[...])
         + amp * dot(g, pamp_ref[...])
         + att * dot(g, patt_ref[...])
         + bpost_ref[...])
    y = dot(y, wlin_ref[...]) + blin_ref[...]
    if do_bn_relu:
        scale = gamma_ref[...] * (1.0 / np.sqrt(1.0 + BN_EPS))
        y = jnp.maximum(y * scale + beta_ref[...], 0.0)
    if do_logsoftmax:
        y = y - jnp.max(y, axis=1, keepdims=True)
        y = y - jnp.log(jnp.sum(jnp.exp(y), axis=1, keepdims=True))
    out_ref[...] = y


def _post_stage(sumD, sqD, mxD, mnD, deg, base, x, w, gamma, beta,
                do_bn_relu, do_logsoftmax):
    N = x.shape[0]
    W, f_in, f_out = w['W'], w['f_in'], w['f_out']
    nb = lambda width: pl.BlockSpec((NODE_BLK, width), lambda i: (i, 0))
    full = lambda a, b: pl.BlockSpec((a, b), lambda i: (0, 0))
    return pl.pallas_call(
        partial(_post_kernel, do_bn_relu=do_bn_relu,
                do_logsoftmax=do_logsoftmax),
        grid=(N // NODE_BLK,),
        in_specs=[nb(W), nb(W), nb(W), nb(W), nb(1), nb(W), nb(f_in),
                  full(f_in, f_out), full(4 * W, f_out), full(4 * W, f_out),
                  full(4 * W, f_out), full(f_out, f_out),
                  full(1, f_out), full(1, f_out), full(1, f_out),
                  full(1, f_out)],
        out_specs=nb(f_out),
        out_shape=jax.ShapeDtypeStruct((N, f_out), jnp.float32),
    )(sumD, sqD, mxD, mnD, deg.reshape(N, 1), base, x,
      w['px'], w['p_id'], w['p_amp'], w['p_att'], w['W_lin'],
      w['b_post'].reshape(1, f_out), w['b_lin'].reshape(1, f_out),
      gamma.reshape(1, f_out), beta.reshape(1, f_out))


def _conv_layer(x_pad, ea_pad, src_s, perm_s, dst_s, off, deg, w,
                gamma, beta, do_bn_relu, do_logsoftmax):
    base, B = _node_mm(x_pad, w)
    C = _edge_mm(ea_pad, w)
    sumD, sqD, mxD, mnD = _sc_segment_stats(B, C, src_s, perm_s, dst_s,
                                            off, w['W_sc'])
    if w['W_sc'] != w['W']:
        W = w['W']
        sumD, sqD, mxD, mnD = (sumD[:, :W], sqD[:, :W],
                               mxD[:, :W], mnD[:, :W])
    return _post_stage(sumD, sqD, mxD, mnD, deg, base, x_pad, w,
                       gamma, beta, do_bn_relu, do_logsoftmax)


def kernel(x, edge_index, edge_attr, params):
    src = edge_index[0].astype(jnp.int32)
    dst = edge_index[1].astype(jnp.int32)
    E = src.shape[0]
    f0 = x.shape[1]
    c0p, c1p, c2p = params['conv0'], params['conv1'], params['conv_out']
    f_hid = c1p['W_lin'].shape[0]
    n_cls = c2p['W_lin'].shape[0]
    w0 = _prep_conv(c0p, NUM_TOWERS_HID, f0, f_hid)
    w1 = _prep_conv(c1p, NUM_TOWERS_HID, f_hid, f_hid)
    w2 = _prep_conv(c2p, 1, f_hid, n_cls)

    e_pad = ((E + K_EDGE + EDGE_BLK - 1) // EDGE_BLK) * EDGE_BLK
    eid = jnp.arange(E, dtype=jnp.int32)
    dst_s, src_s, perm = lax.sort((dst, src, eid), num_keys=1)
    pad = lambda a: jnp.concatenate(
        [a, jnp.zeros((e_pad - E,), jnp.int32)])
    src_sp, perm_sp, dst_sp = pad(src_s), pad(perm), pad(dst_s)
    off = jnp.searchsorted(dst_s, jnp.arange(OFF_LEN, dtype=jnp.int32),
                           side='left').astype(jnp.int32)
    deg = (off[1:N_PAD + 1] - off[:N_PAD]).astype(jnp.float32)

    x_pad = jnp.concatenate(
        [x, jnp.zeros((N_PAD - N_NODES, f0), jnp.float32)])
    ea_pad = jnp.concatenate(
        [edge_attr, jnp.zeros((e_pad - E, D_EDGE), jnp.float32)])

    h = _conv_layer(x_pad, ea_pad, src_sp, perm_sp, dst_sp, off, deg, w0,
                    params['bn0']['gamma'], params['bn0']['beta'],
                    True, False)
    h = _conv_layer(h, ea_pad, src_sp, perm_sp, dst_sp, off, deg, w1,
                    params['bn1']['gamma'], params['bn1']['beta'],
                    True, False)
    ones = jnp.ones((n_cls,), jnp.float32)
    h = _conv_layer(h, ea_pad, src_sp, perm_sp, dst_sp, off, deg, w2,
                    ones, ones * 0.0, False, True)
    return h[:N_NODES]


# speedup vs baseline: 37.8746x; 1.0036x over previous
import functools
from functools import partial

import jax
import jax.numpy as jnp
import numpy as np
from jax import lax
from jax.experimental import pallas as pl
from jax.experimental.pallas import tpu as pltpu
from jax.experimental.pallas import tpu_sc as plsc

N_NODES = 10000
D_EDGE = 16
NUM_TOWERS_HID = 4
AVG_DEG_LOG = float(np.log(33.0))
BN_EPS = 1e-5

N_PAD = 10752
NODES_PER_RANGE = 112
RANGES_PER_TILE = 3
N_TILES = 32
OFF_LEN = N_PAD + 16
EDGE_BLK = 2048
K_EDGE = 128
NODE_BLK = 512
NEG_INIT = float('-inf')


def _prep_conv(p, towers, f_in, f_out):
    W = towers * f_in
    W_pre = p['W_pre']
    w1 = jnp.transpose(W_pre[:, :f_in, :], (1, 0, 2)).reshape(f_in, W)
    w2 = jnp.transpose(W_pre[:, f_in:2 * f_in, :], (1, 0, 2)).reshape(f_in, W)
    w3 = jnp.transpose(W_pre[:, 2 * f_in:, :], (1, 0, 2)).reshape(f_in, W)
    wc = p['W_edge'] @ w3
    c0 = p['b_edge'] @ w3
    b_pre = p['b_pre'].reshape(W)

    W_post = p['W_post']
    px = jnp.transpose(W_post[:, :f_in, :], (1, 0, 2)).reshape(f_in, f_out)

    def blockdiag(k):
        blk = W_post[:, (1 + k) * f_in:(2 + k) * f_in, :]
        eye = jnp.eye(towers, dtype=blk.dtype)
        return jnp.einsum('tio,ts->tiso', blk, eye).reshape(W, f_out)

    p_id = jnp.concatenate([blockdiag(k) for k in range(4)], axis=0)
    p_amp = jnp.concatenate([blockdiag(4 + k) for k in range(4)], axis=0)
    p_att = jnp.concatenate([blockdiag(8 + k) for k in range(4)], axis=0)
    W_sc = max(W, 128)
    if W_sc != W:
        padw = lambda a: jnp.concatenate(
            [a, jnp.zeros((a.shape[0], W_sc - W), a.dtype)], axis=1)
        w2, wc = padw(w2), padw(wc)
        c0 = jnp.concatenate([c0, jnp.zeros((W_sc - W,), c0.dtype)])
    return dict(w1=w1, w2=w2, wc=wc, c0=c0, b_pre=b_pre,
                px=px, p_id=p_id, p_amp=p_amp, p_att=p_att,
                b_post=p['b_post'].reshape(f_out),
                W_lin=p['W_lin'], b_lin=p['b_lin'],
                W=W, W_sc=W_sc, f_in=f_in, f_out=f_out)


def _node_mm_kernel(x_ref, w1_ref, w2_ref, bpre_ref, base_ref, b_ref):
    x = x_ref[...]
    base_ref[...] = jnp.dot(x, w1_ref[...],
                            preferred_element_type=jnp.float32) + bpre_ref[...]
    b_ref[...] = jnp.dot(x, w2_ref[...], preferred_element_type=jnp.float32)


def _node_mm(x, w):
    n = x.shape[0]
    f_in, W, W_sc = w['f_in'], w['W'], w['W_sc']
    nb = lambda width: pl.BlockSpec((NODE_BLK, width), lambda i: (i, 0))
    full = lambda a, b: pl.BlockSpec((a, b), lambda i: (0, 0))
    return pl.pallas_call(
        _node_mm_kernel,
        grid=(n // NODE_BLK,),
        in_specs=[nb(f_in), full(f_in, W), full(f_in, W_sc), full(1, W)],
        out_specs=[nb(W), nb(W_sc)],
        out_shape=[jax.ShapeDtypeStruct((n, W), jnp.float32),
                   jax.ShapeDtypeStruct((n, W_sc), jnp.float32)],
    )(x, w['w1'], w['w2'], w['b_pre'].reshape(1, W))


def _edge_mm_kernel(ea_ref, wc_ref, c0_ref, c_ref):
    c_ref[...] = jnp.dot(ea_ref[...], wc_ref[...],
                         preferred_element_type=jnp.float32) + c0_ref[...]


def _edge_mm(ea_pad, w):
    e_pad = ea_pad.shape[0]
    W = w['W_sc']
    return pl.pallas_call(
        _edge_mm_kernel,
        grid=(e_pad // EDGE_BLK,),
        in_specs=[pl.BlockSpec((EDGE_BLK, D_EDGE), lambda i: (i, 0)),
                  pl.BlockSpec((D_EDGE, W), lambda i: (0, 0)),
                  pl.BlockSpec((1, W), lambda i: (0, 0))],
        out_specs=pl.BlockSpec((EDGE_BLK, W), lambda i: (i, 0)),
        out_shape=jax.ShapeDtypeStruct((e_pad, W), jnp.float32),
    )(ea_pad, w['wc'], w['c0'].reshape(1, W))


def _sc_segment_stats(B, C, src_s, perm_s, dst_s, off, W):
    WC = min(W, 128)
    n_cc = W // WC
    n_g = WC // 16
    NT = NODES_PER_RANGE + 1
    full_w = (n_cc == 1)

    mesh = plsc.VectorSubcoreMesh(core_axis_name="c", subcore_axis_name="s")

    @functools.partial(
        pl.kernel, mesh=mesh,
        out_type=[jax.ShapeDtypeStruct((N_PAD, W), jnp.float32)] * 4,
        scratch_types=[
            pltpu.VMEM((NODES_PER_RANGE + 16,), jnp.int32),
            pltpu.VMEM((2, K_EDGE), jnp.int32),
            pltpu.VMEM((2, K_EDGE), jnp.int32),
            pltpu.VMEM((2, K_EDGE), jnp.int32),
            pltpu.SMEM((2, K_EDGE), jnp.int32),
            pltpu.VMEM((2, K_EDGE, WC), jnp.float32),
            pltpu.VMEM((2, K_EDGE, WC), jnp.float32),
            pltpu.VMEM((NT, WC), jnp.float32),
            pltpu.VMEM((NT, WC), jnp.float32),
            pltpu.VMEM((NT, WC), jnp.float32),
            pltpu.VMEM((NT, WC), jnp.float32),
            pltpu.SemaphoreType.DMA,
            pltpu.SemaphoreType.DMA,
            pltpu.SemaphoreType.DMA,
            pltpu.SemaphoreType.DMA,
        ],
    )
    def seg_kernel(b_h, c_h, src_h, perm_h, dst_h, off_h,
                   o_sum, o_sq, o_mx, o_mn,
                   offv, idxv, permv, dstv, rowsv, bbuf, cbuf,
                   asum, asq, amx, amn, sem_b0, sem_b1, sem_c0, sem_c1):
        cid = lax.axis_index("c")
        sid = lax.axis_index("s")
        t = sid * 2 + cid
        sem_b = (sem_b0, sem_b1)
        sem_c = (sem_c0, sem_c1)
        lane = lax.iota(jnp.int32, 16)

        def do_range(r, _):
            rng = t + r * N_TILES
            n0 = pl.multiple_of(rng * NODES_PER_RANGE, 8)
            pltpu.sync_copy(off_h.at[pl.ds(n0, NODES_PER_RANGE + 16)], offv)
            e0 = offv[pl.ds(0, 16)][0]
            e1 = offv[pl.ds(NODES_PER_RANGE, 16)][0]
            e0a = lax.bitwise_and(e0, jnp.int32(-8))
            nch = lax.div(e1 - e0a + jnp.int32(K_EDGE - 1),
                          jnp.int32(K_EDGE))

            def col_pass(cc, _):
                col0 = pl.multiple_of(cc * WC, 128)
                csl = slice(None) if full_w else pl.ds(col0, WC)

                def load_chunk(i, b):
                    base = pl.multiple_of(e0a + i * K_EDGE, 8)
                    pltpu.sync_copy(src_h.at[pl.ds(base, K_EDGE)],
                                    idxv.at[b])
                    pltpu.sync_copy(perm_h.at[pl.ds(base, K_EDGE)],
                                    permv.at[b])
                    pltpu.sync_copy(dst_h.at[pl.ds(base, K_EDGE)],
                                    dstv.at[b])
                    pltpu.async_copy(b_h.at[idxv.at[b], csl], bbuf.at[b],
                                     sem_b[b])
                    pltpu.async_copy(c_h.at[permv.at[b], csl], cbuf.at[b],
                                     sem_c[b])
                    for j in range(K_EDGE // 16):
                        eidx = base + j * 16 + lane
                        dv = dstv[b, pl.ds(j * 16, 16)]
                        ok = (eidx >= e0) & (eidx < e1)
                        rows = jnp.where(ok, dv - n0, jnp.int32(NT - 1))
                        for k in range(16):
                            rowsv[b, j * 16 + k] = rows[k]

                def init_row(r, _):
                    for g in range(n_g):
                        sl = pl.ds(g * 16, 16)
                        asum[r, sl] = jnp.zeros((16,), jnp.float32)
                        asq[r, sl] = jnp.zeros((16,), jnp.float32)
                        amx[r, sl] = jnp.full((16,), NEG_INIT, jnp.float32)
                        amn[r, sl] = jnp.full((16,), -NEG_INIT, jnp.float32)
                    return 0
                lax.fori_loop(0, NT, init_row, 0)

                @pl.when(nch > 0)
                def _():
                    load_chunk(0, 0)

                def outer(i2, _):
                    for b in range(2):
                        i = i2 * 2 + b

                        @pl.when(i < nch)
                        def _():
                            @pl.when(i + 1 < nch)
                            def _():
                                load_chunk(i + 1, (b + 1) % 2)
                            pltpu.make_async_copy(
                                b_h.at[idxv.at[b], csl], bbuf.at[b],
                                sem_b[b]).wait()
                            pltpu.make_async_copy(
                                c_h.at[permv.at[b], csl], cbuf.at[b],
                                sem_c[b]).wait()

                            def acc_body(kb, _):
                                for off in (0, K_EDGE // 2):
                                    ke = kb + off
                                    row = rowsv[b, ke]
                                    for g in range(n_g):
                                        sl = pl.ds(g * 16, 16)
                                        d = (bbuf[b, ke, sl]
                                             + cbuf[b, ke, sl])
                                        plsc.addupdate(asum.at[row, sl], d)
                                        plsc.addupdate(asq.at[row, sl],
                                                       d * d)
                                        amx[row, sl] = jnp.maximum(
                                            amx[row, sl], d)
                                        amn[row, sl] = jnp.minimum(
                                            amn[row, sl], d)
                                return 0
                            lax.fori_loop(0, K_EDGE // 2, acc_body, 0)
                    return 0
                lax.fori_loop(0, lax.div(nch + 1, jnp.int32(2)), outer, 0)

                nsl = pl.ds(n0, NODES_PER_RANGE)
                src_sl = pl.ds(0, NODES_PER_RANGE)
                pltpu.sync_copy(asum.at[src_sl, :], o_sum.at[nsl, csl])
                pltpu.sync_copy(asq.at[src_sl, :], o_sq.at[nsl, csl])
                pltpu.sync_copy(amx.at[src_sl, :], o_mx.at[nsl, csl])
                pltpu.sync_copy(amn.at[src_sl, :], o_mn.at[nsl, csl])
                return 0
            lax.fori_loop(0, n_cc, col_pass, 0)
            return 0
        lax.fori_loop(0, RANGES_PER_TILE, do_range, 0)

    return seg_kernel(B, C, src_s, perm_s, dst_s, off)


def _post_kernel(sum_ref, sq_ref, mx_ref, mn_ref, deg_ref, base_ref, x_ref,
                 px_ref, pid_ref, pamp_ref, patt_ref, wlin_ref,
                 bpost_ref, blin_ref, gamma_ref, beta_ref,
                 out_ref, *, do_bn_relu, do_logsoftmax):
    deg = deg_ref[...]
    base = base_ref[...]
    deg_safe = jnp.maximum(deg, 1.0)
    inv = 1.0 / deg_safe
    has = deg > 0.0
    meanD = sum_ref[...] * inv
    mean = jnp.where(has, base + meanD, 0.0)
    var = sq_ref[...] * inv - meanD * meanD
    std = jnp.sqrt(jnp.maximum(var, 0.0) + 1e-5)
    mx = jnp.where(has, base + mx_ref[...], 0.0)
    mn = jnp.where(has, base + mn_ref[...], 0.0)
    g = jnp.concatenate([mean, mn, mx, std], axis=1)
    logd = jnp.log(deg + 1.0)
    amp = logd * (1.0 / AVG_DEG_LOG)
    att = jnp.where(has, AVG_DEG_LOG / jnp.maximum(logd, 1e-12), 1.0)
    dot = lambda a, b: jnp.dot(a, b, preferred_element_type=jnp.float32)
    y = (dot(x_ref[...], px_ref[...])
         + dot(g, pid_ref[...])
         + amp * dot(g, pamp_ref[...])
         + att * dot(g, patt_ref[...])
         + bpost_ref[...])
    y = dot(y, wlin_ref[...]) + blin_ref[...]
    if do_bn_relu:
        scale = gamma_ref[...] * (1.0 / np.sqrt(1.0 + BN_EPS))
        y = jnp.maximum(y * scale + beta_ref[...], 0.0)
    if do_logsoftmax:
        y = y - jnp.max(y, axis=1, keepdims=True)
        y = y - jnp.log(jnp.sum(jnp.exp(y), axis=1, keepdims=True))
    out_ref[...] = y


def _post_stage(sumD, sqD, mxD, mnD, deg, base, x, w, gamma, beta,
                do_bn_relu, do_logsoftmax):
    N = x.shape[0]
    W, f_in, f_out = w['W'], w['f_in'], w['f_out']
    nb = lambda width: pl.BlockSpec((NODE_BLK, width), lambda i: (i, 0))
    full = lambda a, b: pl.BlockSpec((a, b), lambda i: (0, 0))
    return pl.pallas_call(
        partial(_post_kernel, do_bn_relu=do_bn_relu,
                do_logsoftmax=do_logsoftmax),
        grid=(N // NODE_BLK,),
        in_specs=[nb(W), nb(W), nb(W), nb(W), nb(1), nb(W), nb(f_in),
                  full(f_in, f_out), full(4 * W, f_out), full(4 * W, f_out),
                  full(4 * W, f_out), full(f_out, f_out),
                  full(1, f_out), full(1, f_out), full(1, f_out),
                  full(1, f_out)],
        out_specs=nb(f_out),
        out_shape=jax.ShapeDtypeStruct((N, f_out), jnp.float32),
    )(sumD, sqD, mxD, mnD, deg.reshape(N, 1), base, x,
      w['px'], w['p_id'], w['p_amp'], w['p_att'], w['W_lin'],
      w['b_post'].reshape(1, f_out), w['b_lin'].reshape(1, f_out),
      gamma.reshape(1, f_out), beta.reshape(1, f_out))


def _conv_layer(x_pad, ea_pad, src_s, perm_s, dst_s, off, deg, w,
                gamma, beta, do_bn_relu, do_logsoftmax):
    base, B = _node_mm(x_pad, w)
    C = _edge_mm(ea_pad, w)
    sumD, sqD, mxD, mnD = _sc_segment_stats(B, C, src_s, perm_s, dst_s,
                                            off, w['W_sc'])
    if w['W_sc'] != w['W']:
        W = w['W']
        sumD, sqD, mxD, mnD = (sumD[:, :W], sqD[:, :W],
                               mxD[:, :W], mnD[:, :W])
    return _post_stage(sumD, sqD, mxD, mnD, deg, base, x_pad, w,
                       gamma, beta, do_bn_relu, do_logsoftmax)


def kernel(x, edge_index, edge_attr, params):
    src = edge_index[0].astype(jnp.int32)
    dst = edge_index[1].astype(jnp.int32)
    E = src.shape[0]
    f0 = x.shape[1]
    c0p, c1p, c2p = params['conv0'], params['conv1'], params['conv_out']
    f_hid = c1p['W_lin'].shape[0]
    n_cls = c2p['W_lin'].shape[0]
    w0 = _prep_conv(c0p, NUM_TOWERS_HID, f0, f_hid)
    w1 = _prep_conv(c1p, NUM_TOWERS_HID, f_hid, f_hid)
    w2 = _prep_conv(c2p, 1, f_hid, n_cls)

    e_pad = ((E + K_EDGE + EDGE_BLK - 1) // EDGE_BLK) * EDGE_BLK
    eid = jnp.arange(E, dtype=jnp.int32)
    dst_s, src_s, perm = lax.sort((dst, src, eid), num_keys=1)
    pad = lambda a: jnp.concatenate(
        [a, jnp.zeros((e_pad - E,), jnp.int32)])
    src_sp, perm_sp, dst_sp = pad(src_s), pad(perm), pad(dst_s)
    off = jnp.searchsorted(dst_s, jnp.arange(OFF_LEN, dtype=jnp.int32),
                           side='left').astype(jnp.int32)
    deg = (off[1:N_PAD + 1] - off[:N_PAD]).astype(jnp.float32)

    x_pad = jnp.concatenate(
        [x, jnp.zeros((N_PAD - N_NODES, f0), jnp.float32)])
    ea_pad = jnp.concatenate(
        [edge_attr, jnp.zeros((e_pad - E, D_EDGE), jnp.float32)])

    h = _conv_layer(x_pad, ea_pad, src_sp, perm_sp, dst_sp, off, deg, w0,
                    params['bn0']['gamma'], params['bn0']['beta'],
                    True, False)
    h = _conv_layer(h, ea_pad, src_sp, perm_sp, dst_sp, off, deg, w1,
                    params['bn1']['gamma'], params['bn1']['beta'],
                    True, False)
    ones = jnp.ones((n_cls,), jnp.float32)
    h = _conv_layer(h, ea_pad, src_sp, perm_sp, dst_sp, off, deg, w2,
                    ones, ones * 0.0, False, True)
    return h[:N_NODES]


# speedup vs baseline: 40.7937x; 1.0771x over previous
import functools
from functools import partial

import jax
import jax.numpy as jnp
import numpy as np
from jax import lax
from jax.experimental import pallas as pl
from jax.experimental.pallas import tpu as pltpu
from jax.experimental.pallas import tpu_sc as plsc

N_NODES = 10000
D_EDGE = 16
NUM_TOWERS_HID = 4
AVG_DEG_LOG = float(np.log(33.0))
BN_EPS = 1e-5

N_PAD = 10752
NODES_PER_RANGE = 112
RANGES_PER_TILE = 3
N_TILES = 32
OFF_LEN = N_PAD + 16
EDGE_BLK = 2048
K_EDGE = 128
NODE_BLK = 512
NEG_INIT = float('-inf')


def _prep_conv(p, towers, f_in, f_out):
    W = towers * f_in
    W_pre = p['W_pre']
    w1 = jnp.transpose(W_pre[:, :f_in, :], (1, 0, 2)).reshape(f_in, W)
    w2 = jnp.transpose(W_pre[:, f_in:2 * f_in, :], (1, 0, 2)).reshape(f_in, W)
    w3 = jnp.transpose(W_pre[:, 2 * f_in:, :], (1, 0, 2)).reshape(f_in, W)
    wc = p['W_edge'] @ w3
    c0 = p['b_edge'] @ w3
    b_pre = p['b_pre'].reshape(W)

    W_post = p['W_post']
    px = jnp.transpose(W_post[:, :f_in, :], (1, 0, 2)).reshape(f_in, f_out)

    def blockdiag(k):
        blk = W_post[:, (1 + k) * f_in:(2 + k) * f_in, :]
        eye = jnp.eye(towers, dtype=blk.dtype)
        return jnp.einsum('tio,ts->tiso', blk, eye).reshape(W, f_out)

    p_id = jnp.concatenate([blockdiag(k) for k in range(4)], axis=0)
    p_amp = jnp.concatenate([blockdiag(4 + k) for k in range(4)], axis=0)
    p_att = jnp.concatenate([blockdiag(8 + k) for k in range(4)], axis=0)
    W_sc = max(W, 128)
    if W_sc != W:
        padw = lambda a: jnp.concatenate(
            [a, jnp.zeros((a.shape[0], W_sc - W), a.dtype)], axis=1)
        w2, wc = padw(w2), padw(wc)
        c0 = jnp.concatenate([c0, jnp.zeros((W_sc - W,), c0.dtype)])
    return dict(w1=w1, w2=w2, wc=wc, c0=c0, b_pre=b_pre,
                px=px, p_id=p_id, p_amp=p_amp, p_att=p_att,
                b_post=p['b_post'].reshape(f_out),
                W_lin=p['W_lin'], b_lin=p['b_lin'],
                W=W, W_sc=W_sc, f_in=f_in, f_out=f_out)


def _node_mm_kernel(x_ref, w1_ref, w2_ref, bpre_ref, base_ref, b_ref):
    x = x_ref[...]
    base_ref[...] = jnp.dot(x, w1_ref[...],
                            preferred_element_type=jnp.float32) + bpre_ref[...]
    b_ref[...] = jnp.dot(x, w2_ref[...], preferred_element_type=jnp.float32)


def _node_mm(x, w):
    n = x.shape[0]
    f_in, W, W_sc = w['f_in'], w['W'], w['W_sc']
    nb = lambda width: pl.BlockSpec((NODE_BLK, width), lambda i: (i, 0))
    full = lambda a, b: pl.BlockSpec((a, b), lambda i: (0, 0))
    return pl.pallas_call(
        _node_mm_kernel,
        grid=(n // NODE_BLK,),
        in_specs=[nb(f_in), full(f_in, W), full(f_in, W_sc), full(1, W)],
        out_specs=[nb(W), nb(W_sc)],
        out_shape=[jax.ShapeDtypeStruct((n, W), jnp.float32),
                   jax.ShapeDtypeStruct((n, W_sc), jnp.float32)],
    )(x, w['w1'], w['w2'], w['b_pre'].reshape(1, W))


def _edge_mm_kernel(ea_ref, wc_ref, c0_ref, c_ref):
    c_ref[...] = jnp.dot(ea_ref[...], wc_ref[...],
                         preferred_element_type=jnp.float32) + c0_ref[...]


def _edge_mm(ea_pad, w):
    e_pad = ea_pad.shape[0]
    W = w['W_sc']
    return pl.pallas_call(
        _edge_mm_kernel,
        grid=(e_pad // EDGE_BLK,),
        in_specs=[pl.BlockSpec((EDGE_BLK, D_EDGE), lambda i: (i, 0)),
                  pl.BlockSpec((D_EDGE, W), lambda i: (0, 0)),
                  pl.BlockSpec((1, W), lambda i: (0, 0))],
        out_specs=pl.BlockSpec((EDGE_BLK, W), lambda i: (i, 0)),
        out_shape=jax.ShapeDtypeStruct((e_pad, W), jnp.float32),
    )(ea_pad, w['wc'], w['c0'].reshape(1, W))


def _sc_segment_stats(B, C, src_s, perm_s, dst_s, off, W):
    WC = min(W, 128)
    n_cc = W // WC
    n_g = WC // 16
    NT = NODES_PER_RANGE + 1
    full_w = (n_cc == 1)

    mesh = plsc.VectorSubcoreMesh(core_axis_name="c", subcore_axis_name="s")

    @functools.partial(
        pl.kernel, mesh=mesh,
        out_type=[jax.ShapeDtypeStruct((N_PAD, W), jnp.float32)] * 4,
        scratch_types=[
            pltpu.VMEM((NODES_PER_RANGE + 16,), jnp.int32),
            pltpu.VMEM((2, K_EDGE), jnp.int32),
            pltpu.VMEM((2, K_EDGE), jnp.int32),
            pltpu.VMEM((2, K_EDGE), jnp.int32),
            pltpu.SMEM((2, K_EDGE), jnp.int32),
            pltpu.VMEM((2, K_EDGE, WC), jnp.float32),
            pltpu.VMEM((2, K_EDGE, WC), jnp.float32),
            pltpu.VMEM((NT, WC), jnp.float32),
            pltpu.VMEM((NT, WC), jnp.float32),
            pltpu.VMEM((NT, WC), jnp.float32),
            pltpu.VMEM((NT, WC), jnp.float32),
            pltpu.SemaphoreType.DMA,
            pltpu.SemaphoreType.DMA,
            pltpu.SemaphoreType.DMA,
            pltpu.SemaphoreType.DMA,
            pltpu.SemaphoreType.DMA,
            pltpu.SemaphoreType.DMA,
        ],
    )
    def seg_kernel(b_h, c_h, src_h, perm_h, dst_h, off_h,
                   o_sum, o_sq, o_mx, o_mn,
                   offv, idxv, permv, dstv, rowsv, bbuf, cbuf,
                   asum, asq, amx, amn, sem_b0, sem_b1, sem_c0, sem_c1,
                   sem_i0, sem_i1):
        cid = lax.axis_index("c")
        sid = lax.axis_index("s")
        t = sid * 2 + cid
        sem_b = (sem_b0, sem_b1)
        sem_c = (sem_c0, sem_c1)
        sem_i = (sem_i0, sem_i1)
        lane = lax.iota(jnp.int32, 16)

        def do_range(r, _):
            rng = t + r * N_TILES
            n0 = pl.multiple_of(rng * NODES_PER_RANGE, 8)
            pltpu.sync_copy(off_h.at[pl.ds(n0, NODES_PER_RANGE + 16)], offv)
            e0 = offv[pl.ds(0, 16)][0]
            e1 = offv[pl.ds(NODES_PER_RANGE, 16)][0]
            e0a = lax.bitwise_and(e0, jnp.int32(-8))
            nch = lax.div(e1 - e0a + jnp.int32(K_EDGE - 1),
                          jnp.int32(K_EDGE))

            def col_pass(cc, _):
                col0 = pl.multiple_of(cc * WC, 128)
                csl = slice(None) if full_w else pl.ds(col0, WC)

                def load_chunk(i, b):
                    base = pl.multiple_of(e0a + i * K_EDGE, 8)
                    c1 = pltpu.async_copy(src_h.at[pl.ds(base, K_EDGE)],
                                          idxv.at[b], sem_i[b])
                    c2 = pltpu.async_copy(perm_h.at[pl.ds(base, K_EDGE)],
                                          permv.at[b], sem_i[b])
                    c3 = pltpu.async_copy(dst_h.at[pl.ds(base, K_EDGE)],
                                          dstv.at[b], sem_i[b])
                    c1.wait()
                    c2.wait()
                    c3.wait()
                    pltpu.async_copy(b_h.at[idxv.at[b], csl], bbuf.at[b],
                                     sem_b[b])
                    pltpu.async_copy(c_h.at[permv.at[b], csl], cbuf.at[b],
                                     sem_c[b])
                    for j in range(K_EDGE // 16):
                        eidx = base + j * 16 + lane
                        dv = dstv[b, pl.ds(j * 16, 16)]
                        ok = (eidx >= e0) & (eidx < e1)
                        rows = jnp.where(ok, dv - n0, jnp.int32(NT - 1))
                        for k in range(16):
                            rowsv[b, j * 16 + k] = rows[k]

                def init_row(r, _):
                    for g in range(n_g):
                        sl = pl.ds(g * 16, 16)
                        asum[r, sl] = jnp.zeros((16,), jnp.float32)
                        asq[r, sl] = jnp.zeros((16,), jnp.float32)
                        amx[r, sl] = jnp.full((16,), NEG_INIT, jnp.float32)
                        amn[r, sl] = jnp.full((16,), -NEG_INIT, jnp.float32)
                    return 0
                lax.fori_loop(0, NT, init_row, 0)

                @pl.when(nch > 0)
                def _():
                    load_chunk(0, 0)

                def outer(i2, _):
                    for b in range(2):
                        i = i2 * 2 + b

                        @pl.when(i < nch)
                        def _():
                            @pl.when(i + 1 < nch)
                            def _():
                                load_chunk(i + 1, (b + 1) % 2)
                            pltpu.make_async_copy(
                                b_h.at[idxv.at[b], csl], bbuf.at[b],
                                sem_b[b]).wait()
                            pltpu.make_async_copy(
                                c_h.at[permv.at[b], csl], cbuf.at[b],
                                sem_c[b]).wait()

                            def acc_body(kb, _):
                                for off in (0, K_EDGE // 2):
                                    ke = kb + off
                                    row = rowsv[b, ke]
                                    for g in range(n_g):
                                        sl = pl.ds(g * 16, 16)
                                        d = (bbuf[b, ke, sl]
                                             + cbuf[b, ke, sl])
                                        plsc.addupdate(asum.at[row, sl], d)
                                        plsc.addupdate(asq.at[row, sl],
                                                       d * d)
                                        amx[row, sl] = jnp.maximum(
                                            amx[row, sl], d)
                                        amn[row, sl] = jnp.minimum(
                                            amn[row, sl], d)
                                return 0
                            lax.fori_loop(0, K_EDGE // 2, acc_body, 0)
                    return 0
                lax.fori_loop(0, lax.div(nch + 1, jnp.int32(2)), outer, 0)

                nsl = pl.ds(n0, NODES_PER_RANGE)
                src_sl = pl.ds(0, NODES_PER_RANGE)
                pltpu.sync_copy(asum.at[src_sl, :], o_sum.at[nsl, csl])
                pltpu.sync_copy(asq.at[src_sl, :], o_sq.at[nsl, csl])
                pltpu.sync_copy(amx.at[src_sl, :], o_mx.at[nsl, csl])
                pltpu.sync_copy(amn.at[src_sl, :], o_mn.at[nsl, csl])
                return 0
            lax.fori_loop(0, n_cc, col_pass, 0)
            return 0
        lax.fori_loop(0, RANGES_PER_TILE, do_range, 0)

    return seg_kernel(B, C, src_s, perm_s, dst_s, off)


def _post_kernel(sum_ref, sq_ref, mx_ref, mn_ref, deg_ref, base_ref, x_ref,
                 px_ref, pid_ref, pamp_ref, patt_ref, wlin_ref,
                 bpost_ref, blin_ref, gamma_ref, beta_ref,
                 out_ref, *, do_bn_relu, do_logsoftmax):
    deg = deg_ref[...]
    base = base_ref[...]
    deg_safe = jnp.maximum(deg, 1.0)
    inv = 1.0 / deg_safe
    has = deg > 0.0
    meanD = sum_ref[...] * inv
    mean = jnp.where(has, base + meanD, 0.0)
    var = sq_ref[...] * inv - meanD * meanD
    std = jnp.sqrt(jnp.maximum(var, 0.0) + 1e-5)
    mx = jnp.where(has, base + mx_ref[...], 0.0)
    mn = jnp.where(has, base + mn_ref[...], 0.0)
    g = jnp.concatenate([mean, mn, mx, std], axis=1)
    logd = jnp.log(deg + 1.0)
    amp = logd * (1.0 / AVG_DEG_LOG)
    att = jnp.where(has, AVG_DEG_LOG / jnp.maximum(logd, 1e-12), 1.0)
    dot = lambda a, b: jnp.dot(a, b, preferred_element_type=jnp.float32)
    y = (dot(x_ref[...], px_ref[...])
         + dot(g, pid_ref[...])
         + amp * dot(g, pamp_ref[...])
         + att * dot(g, patt_ref[...])
         + bpost_ref[...])
    y = dot(y, wlin_ref[...]) + blin_ref[...]
    if do_bn_relu:
        scale = gamma_ref[...] * (1.0 / np.sqrt(1.0 + BN_EPS))
        y = jnp.maximum(y * scale + beta_ref[...], 0.0)
    if do_logsoftmax:
        y = y - jnp.max(y, axis=1, keepdims=True)
        y = y - jnp.log(jnp.sum(jnp.exp(y), axis=1, keepdims=True))
    out_ref[...] = y


def _post_stage(sumD, sqD, mxD, mnD, deg, base, x, w, gamma, beta,
                do_bn_relu, do_logsoftmax):
    N = x.shape[0]
    W, f_in, f_out = w['W'], w['f_in'], w['f_out']
    nb = lambda width: pl.BlockSpec((NODE_BLK, width), lambda i: (i, 0))
    full = lambda a, b: pl.BlockSpec((a, b), lambda i: (0, 0))
    return pl.pallas_call(
        partial(_post_kernel, do_bn_relu=do_bn_relu,
                do_logsoftmax=do_logsoftmax),
        grid=(N // NODE_BLK,),
        in_specs=[nb(W), nb(W), nb(W), nb(W), nb(1), nb(W), nb(f_in),
                  full(f_in, f_out), full(4 * W, f_out), full(4 * W, f_out),
                  full(4 * W, f_out), full(f_out, f_out),
                  full(1, f_out), full(1, f_out), full(1, f_out),
                  full(1, f_out)],
        out_specs=nb(f_out),
        out_shape=jax.ShapeDtypeStruct((N, f_out), jnp.float32),
    )(sumD, sqD, mxD, mnD, deg.reshape(N, 1), base, x,
      w['px'], w['p_id'], w['p_amp'], w['p_att'], w['W_lin'],
      w['b_post'].reshape(1, f_out), w['b_lin'].reshape(1, f_out),
      gamma.reshape(1, f_out), beta.reshape(1, f_out))


def _conv_layer(x_pad, ea_pad, src_s, perm_s, dst_s, off, deg, w,
                gamma, beta, do_bn_relu, do_logsoftmax):
    base, B = _node_mm(x_pad, w)
    C = _edge_mm(ea_pad, w)
    sumD, sqD, mxD, mnD = _sc_segment_stats(B, C, src_s, perm_s, dst_s,
                                            off, w['W_sc'])
    if w['W_sc'] != w['W']:
        W = w['W']
        sumD, sqD, mxD, mnD = (sumD[:, :W], sqD[:, :W],
                               mxD[:, :W], mnD[:, :W])
    return _post_stage(sumD, sqD, mxD, mnD, deg, base, x_pad, w,
                       gamma, beta, do_bn_relu, do_logsoftmax)


def kernel(x, edge_index, edge_attr, params):
    src = edge_index[0].astype(jnp.int32)
    dst = edge_index[1].astype(jnp.int32)
    E = src.shape[0]
    f0 = x.shape[1]
    c0p, c1p, c2p = params['conv0'], params['conv1'], params['conv_out']
    f_hid = c1p['W_lin'].shape[0]
    n_cls = c2p['W_lin'].shape[0]
    w0 = _prep_conv(c0p, NUM_TOWERS_HID, f0, f_hid)
    w1 = _prep_conv(c1p, NUM_TOWERS_HID, f_hid, f_hid)
    w2 = _prep_conv(c2p, 1, f_hid, n_cls)

    e_pad = ((E + K_EDGE + EDGE_BLK - 1) // EDGE_BLK) * EDGE_BLK
    eid = jnp.arange(E, dtype=jnp.int32)
    dst_s, src_s, perm = lax.sort((dst, src, eid), num_keys=1)
    pad = lambda a: jnp.concatenate(
        [a, jnp.zeros((e_pad - E,), jnp.int32)])
    src_sp, perm_sp, dst_sp = pad(src_s), pad(perm), pad(dst_s)
    off = jnp.searchsorted(dst_s, jnp.arange(OFF_LEN, dtype=jnp.int32),
                           side='left').astype(jnp.int32)
    deg = (off[1:N_PAD + 1] - off[:N_PAD]).astype(jnp.float32)

    x_pad = jnp.concatenate(
        [x, jnp.zeros((N_PAD - N_NODES, f0), jnp.float32)])
    ea_pad = jnp.concatenate(
        [edge_attr, jnp.zeros((e_pad - E, D_EDGE), jnp.float32)])

    h = _conv_layer(x_pad, ea_pad, src_sp, perm_sp, dst_sp, off, deg, w0,
                    params['bn0']['gamma'], params['bn0']['beta'],
                    True, False)
    h = _conv_layer(h, ea_pad, src_sp, perm_sp, dst_sp, off, deg, w1,
                    params['bn1']['gamma'], params['bn1']['beta'],
                    True, False)
    ones = jnp.ones((n_cls,), jnp.float32)
    h = _conv_layer(h, ea_pad, src_sp, perm_sp, dst_sp, off, deg, w2,
                    ones, ones * 0.0, False, True)
    return h[:N_NODES]


# speedup vs baseline: 42.4933x; 1.0417x over previous
import functools
from functools import partial

import jax
import jax.numpy as jnp
import numpy as np
from jax import lax
from jax.experimental import pallas as pl
from jax.experimental.pallas import tpu as pltpu
from jax.experimental.pallas import tpu_sc as plsc

N_NODES = 10000
D_EDGE = 16
NUM_TOWERS_HID = 4
AVG_DEG_LOG = float(np.log(33.0))
BN_EPS = 1e-5

N_PAD = 10752
NODES_PER_RANGE = 112
RANGES_PER_TILE = 3
N_TILES = 32
OFF_LEN = N_PAD + 16
EDGE_BLK = 2048
K_EDGE = 128
NODE_BLK = 512
NEG_INIT = float('-inf')


def _prep_conv(p, towers, f_in, f_out):
    W = towers * f_in
    W_pre = p['W_pre']
    w1 = jnp.transpose(W_pre[:, :f_in, :], (1, 0, 2)).reshape(f_in, W)
    w2 = jnp.transpose(W_pre[:, f_in:2 * f_in, :], (1, 0, 2)).reshape(f_in, W)
    w3 = jnp.transpose(W_pre[:, 2 * f_in:, :], (1, 0, 2)).reshape(f_in, W)
    wc = p['W_edge'] @ w3
    c0 = p['b_edge'] @ w3
    b_pre = p['b_pre'].reshape(W)

    W_post = p['W_post']
    px = jnp.transpose(W_post[:, :f_in, :], (1, 0, 2)).reshape(f_in, f_out)

    def blockdiag(k):
        blk = W_post[:, (1 + k) * f_in:(2 + k) * f_in, :]
        eye = jnp.eye(towers, dtype=blk.dtype)
        return jnp.einsum('tio,ts->tiso', blk, eye).reshape(W, f_out)

    p_id = jnp.concatenate([blockdiag(k) for k in range(4)], axis=0)
    p_amp = jnp.concatenate([blockdiag(4 + k) for k in range(4)], axis=0)
    p_att = jnp.concatenate([blockdiag(8 + k) for k in range(4)], axis=0)
    W_sc = max(W, 128)
    if W_sc != W:
        padw = lambda a: jnp.concatenate(
            [a, jnp.zeros((a.shape[0], W_sc - W), a.dtype)], axis=1)
        w2, wc = padw(w2), padw(wc)
        c0 = jnp.concatenate([c0, jnp.zeros((W_sc - W,), c0.dtype)])
    return dict(w1=w1, w2=w2, wc=wc, c0=c0, b_pre=b_pre,
                px=px, p_id=p_id, p_amp=p_amp, p_att=p_att,
                b_post=p['b_post'].reshape(f_out),
                W_lin=p['W_lin'], b_lin=p['b_lin'],
                W=W, W_sc=W_sc, f_in=f_in, f_out=f_out)


def _node_mm_kernel(x_ref, w1_ref, w2_ref, bpre_ref, base_ref, b_ref):
    x = x_ref[...]
    base_ref[...] = jnp.dot(x, w1_ref[...],
                            preferred_element_type=jnp.float32) + bpre_ref[...]
    b_ref[...] = jnp.dot(x, w2_ref[...], preferred_element_type=jnp.float32)


def _node_mm(x, w):
    n = x.shape[0]
    f_in, W, W_sc = w['f_in'], w['W'], w['W_sc']
    nb = lambda width: pl.BlockSpec((NODE_BLK, width), lambda i: (i, 0))
    full = lambda a, b: pl.BlockSpec((a, b), lambda i: (0, 0))
    return pl.pallas_call(
        _node_mm_kernel,
        grid=(n // NODE_BLK,),
        in_specs=[nb(f_in), full(f_in, W), full(f_in, W_sc), full(1, W)],
        out_specs=[nb(W), nb(W_sc)],
        out_shape=[jax.ShapeDtypeStruct((n, W), jnp.float32),
                   jax.ShapeDtypeStruct((n, W_sc), jnp.float32)],
    )(x, w['w1'], w['w2'], w['b_pre'].reshape(1, W))


def _edge_mm_kernel(ea_ref, wc_ref, c0_ref, c_ref):
    c_ref[...] = jnp.dot(ea_ref[...], wc_ref[...],
                         preferred_element_type=jnp.float32) + c0_ref[...]


def _edge_mm(ea_pad, w):
    e_pad = ea_pad.shape[0]
    W = w['W_sc']
    return pl.pallas_call(
        _edge_mm_kernel,
        grid=(e_pad // EDGE_BLK,),
        in_specs=[pl.BlockSpec((EDGE_BLK, D_EDGE), lambda i: (i, 0)),
                  pl.BlockSpec((D_EDGE, W), lambda i: (0, 0)),
                  pl.BlockSpec((1, W), lambda i: (0, 0))],
        out_specs=pl.BlockSpec((EDGE_BLK, W), lambda i: (i, 0)),
        out_shape=jax.ShapeDtypeStruct((e_pad, W), jnp.float32),
    )(ea_pad, w['wc'], w['c0'].reshape(1, W))


def _sc_segment_stats(B, C, src_s, perm_s, dst_s, off, W):
    WC = min(W, 128)
    n_cc = W // WC
    n_g = WC // 16
    NT = NODES_PER_RANGE + 1
    full_w = (n_cc == 1)

    mesh = plsc.VectorSubcoreMesh(core_axis_name="c", subcore_axis_name="s")

    @functools.partial(
        pl.kernel, mesh=mesh,
        out_type=[jax.ShapeDtypeStruct((N_PAD, W), jnp.float32)] * 4,
        scratch_types=[
            pltpu.VMEM((NODES_PER_RANGE + 16,), jnp.int32),
            pltpu.VMEM((2, K_EDGE), jnp.int32),
            pltpu.VMEM((2, K_EDGE), jnp.int32),
            pltpu.VMEM((2, K_EDGE), jnp.int32),
            pltpu.SMEM((2, K_EDGE), jnp.int32),
            pltpu.VMEM((2, K_EDGE, WC), jnp.float32),
            pltpu.VMEM((2, K_EDGE, WC), jnp.float32),
            pltpu.VMEM((NT, WC), jnp.float32),
            pltpu.VMEM((NT, WC), jnp.float32),
            pltpu.VMEM((NT, WC), jnp.float32),
            pltpu.VMEM((NT, WC), jnp.float32),
            pltpu.SemaphoreType.DMA,
            pltpu.SemaphoreType.DMA,
            pltpu.SemaphoreType.DMA,
            pltpu.SemaphoreType.DMA,
            pltpu.SemaphoreType.DMA,
            pltpu.SemaphoreType.DMA,
        ],
    )
    def seg_kernel(b_h, c_h, src_h, perm_h, dst_h, off_h,
                   o_sum, o_sq, o_mx, o_mn,
                   offv, idxv, permv, dstv, rowsv, bbuf, cbuf,
                   asum, asq, amx, amn, sem_b0, sem_b1, sem_c0, sem_c1,
                   sem_i0, sem_i1):
        cid = lax.axis_index("c")
        sid = lax.axis_index("s")
        t = sid * 2 + cid
        sem_b = (sem_b0, sem_b1)
        sem_c = (sem_c0, sem_c1)
        sem_i = (sem_i0, sem_i1)
        lane = lax.iota(jnp.int32, 16)

        def do_range(r, _):
            rng = t + r * N_TILES
            n0 = pl.multiple_of(rng * NODES_PER_RANGE, 8)
            pltpu.sync_copy(off_h.at[pl.ds(n0, NODES_PER_RANGE + 16)], offv)
            e0 = offv[pl.ds(0, 16)][0]
            e1 = offv[pl.ds(NODES_PER_RANGE, 16)][0]
            e0a = lax.bitwise_and(e0, jnp.int32(-8))
            nch = lax.div(e1 - e0a + jnp.int32(K_EDGE - 1),
                          jnp.int32(K_EDGE))

            def col_pass(cc, _):
                col0 = pl.multiple_of(cc * WC, 128)
                csl = slice(None) if full_w else pl.ds(col0, WC)

                def issue_idx(i, b):
                    base = pl.multiple_of(e0a + i * K_EDGE, 8)
                    pltpu.async_copy(src_h.at[pl.ds(base, K_EDGE)],
                                     idxv.at[b], sem_i[b])
                    pltpu.async_copy(perm_h.at[pl.ds(base, K_EDGE)],
                                     permv.at[b], sem_i[b])
                    pltpu.async_copy(dst_h.at[pl.ds(base, K_EDGE)],
                                     dstv.at[b], sem_i[b])

                def prep_chunk(i, b):
                    base = pl.multiple_of(e0a + i * K_EDGE, 8)
                    pltpu.make_async_copy(src_h.at[pl.ds(base, K_EDGE)],
                                          idxv.at[b], sem_i[b]).wait()
                    pltpu.make_async_copy(perm_h.at[pl.ds(base, K_EDGE)],
                                          permv.at[b], sem_i[b]).wait()
                    pltpu.make_async_copy(dst_h.at[pl.ds(base, K_EDGE)],
                                          dstv.at[b], sem_i[b]).wait()
                    pltpu.async_copy(b_h.at[idxv.at[b], csl], bbuf.at[b],
                                     sem_b[b])
                    pltpu.async_copy(c_h.at[permv.at[b], csl], cbuf.at[b],
                                     sem_c[b])
                    for j in range(K_EDGE // 16):
                        eidx = base + j * 16 + lane
                        dv = dstv[b, pl.ds(j * 16, 16)]
                        ok = (eidx >= e0) & (eidx < e1)
                        rows = jnp.where(ok, dv - n0, jnp.int32(NT - 1))
                        for k in range(16):
                            rowsv[b, j * 16 + k] = rows[k]

                def init_row(r, _):
                    for g in range(n_g):
                        sl = pl.ds(g * 16, 16)
                        asum[r, sl] = jnp.zeros((16,), jnp.float32)
                        asq[r, sl] = jnp.zeros((16,), jnp.float32)
                        amx[r, sl] = jnp.full((16,), NEG_INIT, jnp.float32)
                        amn[r, sl] = jnp.full((16,), -NEG_INIT, jnp.float32)
                    return 0
                lax.fori_loop(0, NT, init_row, 0)

                @pl.when(nch > 0)
                def _():
                    issue_idx(jnp.int32(0), 0)
                    prep_chunk(jnp.int32(0), 0)
                @pl.when(nch > 1)
                def _():
                    issue_idx(jnp.int32(1), 1)

                def outer(i2, _):
                    for b in range(2):
                        i = i2 * 2 + b

                        @pl.when(i < nch)
                        def _():
                            @pl.when(i + 1 < nch)
                            def _():
                                prep_chunk(i + 1, (b + 1) % 2)
                            pltpu.make_async_copy(
                                b_h.at[idxv.at[b], csl], bbuf.at[b],
                                sem_b[b]).wait()
                            pltpu.make_async_copy(
                                c_h.at[permv.at[b], csl], cbuf.at[b],
                                sem_c[b]).wait()

                            @pl.when(i + 2 < nch)
                            def _():
                                issue_idx(i + 2, b)

                            def acc_body(kb, _):
                                for off in (0, K_EDGE // 2):
                                    ke = kb + off
                                    row = rowsv[b, ke]
                                    for g in range(n_g):
                                        sl = pl.ds(g * 16, 16)
                                        d = (bbuf[b, ke, sl]
                                             + cbuf[b, ke, sl])
                                        plsc.addupdate(asum.at[row, sl], d)
                                        plsc.addupdate(asq.at[row, sl],
                                                       d * d)
                                        amx[row, sl] = jnp.maximum(
                                            amx[row, sl], d)
                                        amn[row, sl] = jnp.minimum(
                                            amn[row, sl], d)
                                return 0
                            lax.fori_loop(0, K_EDGE // 2, acc_body, 0)
                    return 0
                lax.fori_loop(0, lax.div(nch + 1, jnp.int32(2)), outer, 0)

                nsl = pl.ds(n0, NODES_PER_RANGE)
                src_sl = pl.ds(0, NODES_PER_RANGE)
                pltpu.sync_copy(asum.at[src_sl, :], o_sum.at[nsl, csl])
                pltpu.sync_copy(asq.at[src_sl, :], o_sq.at[nsl, csl])
                pltpu.sync_copy(amx.at[src_sl, :], o_mx.at[nsl, csl])
                pltpu.sync_copy(amn.at[src_sl, :], o_mn.at[nsl, csl])
                return 0
            lax.fori_loop(0, n_cc, col_pass, 0)
            return 0
        lax.fori_loop(0, RANGES_PER_TILE, do_range, 0)

    return seg_kernel(B, C, src_s, perm_s, dst_s, off)


def _post_kernel(sum_ref, sq_ref, mx_ref, mn_ref, deg_ref, base_ref, x_ref,
                 px_ref, pid_ref, pamp_ref, patt_ref, wlin_ref,
                 bpost_ref, blin_ref, gamma_ref, beta_ref,
                 out_ref, *, do_bn_relu, do_logsoftmax):
    deg = deg_ref[...]
    base = base_ref[...]
    deg_safe = jnp.maximum(deg, 1.0)
    inv = 1.0 / deg_safe
    has = deg > 0.0
    meanD = sum_ref[...] * inv
    mean = jnp.where(has, base + meanD, 0.0)
    var = sq_ref[...] * inv - meanD * meanD
    std = jnp.sqrt(jnp.maximum(var, 0.0) + 1e-5)
    mx = jnp.where(has, base + mx_ref[...], 0.0)
    mn = jnp.where(has, base + mn_ref[...], 0.0)
    g = jnp.concatenate([mean, mn, mx, std], axis=1)
    logd = jnp.log(deg + 1.0)
    amp = logd * (1.0 / AVG_DEG_LOG)
    att = jnp.where(has, AVG_DEG_LOG / jnp.maximum(logd, 1e-12), 1.0)
    dot = lambda a, b: jnp.dot(a, b, preferred_element_type=jnp.float32)
    y = (dot(x_ref[...], px_ref[...])
         + dot(g, pid_ref[...])
         + amp * dot(g, pamp_ref[...])
         + att * dot(g, patt_ref[...])
         + bpost_ref[...])
    y = dot(y, wlin_ref[...]) + blin_ref[...]
    if do_bn_relu:
        scale = gamma_ref[...] * (1.0 / np.sqrt(1.0 + BN_EPS))
        y = jnp.maximum(y * scale + beta_ref[...], 0.0)
    if do_logsoftmax:
        y = y - jnp.max(y, axis=1, keepdims=True)
        y = y - jnp.log(jnp.sum(jnp.exp(y), axis=1, keepdims=True))
    out_ref[...] = y


def _post_stage(sumD, sqD, mxD, mnD, deg, base, x, w, gamma, beta,
                do_bn_relu, do_logsoftmax):
    N = x.shape[0]
    W, f_in, f_out = w['W'], w['f_in'], w['f_out']
    nb = lambda width: pl.BlockSpec((NODE_BLK, width), lambda i: (i, 0))
    full = lambda a, b: pl.BlockSpec((a, b), lambda i: (0, 0))
    return pl.pallas_call(
        partial(_post_kernel, do_bn_relu=do_bn_relu,
                do_logsoftmax=do_logsoftmax),
        grid=(N // NODE_BLK,),
        in_specs=[nb(W), nb(W), nb(W), nb(W), nb(1), nb(W), nb(f_in),
                  full(f_in, f_out), full(4 * W, f_out), full(4 * W, f_out),
                  full(4 * W, f_out), full(f_out, f_out),
                  full(1, f_out), full(1, f_out), full(1, f_out),
                  full(1, f_out)],
        out_specs=nb(f_out),
        out_shape=jax.ShapeDtypeStruct((N, f_out), jnp.float32),
    )(sumD, sqD, mxD, mnD, deg.reshape(N, 1), base, x,
      w['px'], w['p_id'], w['p_amp'], w['p_att'], w['W_lin'],
      w['b_post'].reshape(1, f_out), w['b_lin'].reshape(1, f_out),
      gamma.reshape(1, f_out), beta.reshape(1, f_out))


def _conv_layer(x_pad, ea_pad, src_s, perm_s, dst_s, off, deg, w,
                gamma, beta, do_bn_relu, do_logsoftmax):
    base, B = _node_mm(x_pad, w)
    C = _edge_mm(ea_pad, w)
    sumD, sqD, mxD, mnD = _sc_segment_stats(B, C, src_s, perm_s, dst_s,
                                            off, w['W_sc'])
    if w['W_sc'] != w['W']:
        W = w['W']
        sumD, sqD, mxD, mnD = (sumD[:, :W], sqD[:, :W],
                               mxD[:, :W], mnD[:, :W])
    return _post_stage(sumD, sqD, mxD, mnD, deg, base, x_pad, w,
                       gamma, beta, do_bn_relu, do_logsoftmax)


def kernel(x, edge_index, edge_attr, params):
    src = edge_index[0].astype(jnp.int32)
    dst = edge_index[1].astype(jnp.int32)
    E = src.shape[0]
    f0 = x.shape[1]
    c0p, c1p, c2p = params['conv0'], params['conv1'], params['conv_out']
    f_hid = c1p['W_lin'].shape[0]
    n_cls = c2p['W_lin'].shape[0]
    w0 = _prep_conv(c0p, NUM_TOWERS_HID, f0, f_hid)
    w1 = _prep_conv(c1p, NUM_TOWERS_HID, f_hid, f_hid)
    w2 = _prep_conv(c2p, 1, f_hid, n_cls)

    e_pad = ((E + K_EDGE + EDGE_BLK - 1) // EDGE_BLK) * EDGE_BLK
    eid = jnp.arange(E, dtype=jnp.int32)
    dst_s, src_s, perm = lax.sort((dst, src, eid), num_keys=1)
    pad = lambda a: jnp.concatenate(
        [a, jnp.zeros((e_pad - E,), jnp.int32)])
    src_sp, perm_sp, dst_sp = pad(src_s), pad(perm), pad(dst_s)
    off = jnp.searchsorted(dst_s, jnp.arange(OFF_LEN, dtype=jnp.int32),
                           side='left').astype(jnp.int32)
    deg = (off[1:N_PAD + 1] - off[:N_PAD]).astype(jnp.float32)

    x_pad = jnp.concatenate(
        [x, jnp.zeros((N_PAD - N_NODES, f0), jnp.float32)])
    ea_pad = jnp.concatenate(
        [edge_attr, jnp.zeros((e_pad - E, D_EDGE), jnp.float32)])

    h = _conv_layer(x_pad, ea_pad, src_sp, perm_sp, dst_sp, off, deg, w0,
                    params['bn0']['gamma'], params['bn0']['beta'],
                    True, False)
    h = _conv_layer(h, ea_pad, src_sp, perm_sp, dst_sp, off, deg, w1,
                    params['bn1']['gamma'], params['bn1']['beta'],
                    True, False)
    ones = jnp.ones((n_cls,), jnp.float32)
    h = _conv_layer(h, ea_pad, src_sp, perm_sp, dst_sp, off, deg, w2,
                    ones, ones * 0.0, False, True)
    return h[:N_NODES]
